# baseline (speedup 1.0000x reference)
import functools

import jax
import jax.numpy as jnp
from jax import lax
from jax.experimental import pallas as pl
from jax.experimental.pallas import tpu as pltpu
from jax.experimental.pallas import tpu_sc as plsc

PADW = 16
CHUNK = 64
GRP = 8
NSLAB = 8
SLAB = 1 << 17
BLK = 1024


def _make_fold(E, V):
    K = NSLAB * E
    nstep = SLAB // BLK
    tail0 = (V - (NSLAB - 1) * SLAB) // BLK

    def body(et_hbm, tail_hbm, w2_ref, b2_ref, out_ref, scratch, sems):
        i = pl.program_id(0)

        def piece(step, buf, a):
            dst = scratch.at[buf, pl.ds(a * E, E), :]
            if a == NSLAB - 1:
                def tail_cp():
                    pltpu.make_async_copy(
                        tail_hbm.at[:, pl.ds((step - tail0) * BLK, BLK)],
                        dst, sems.at[buf]).start()

                def main_cp():
                    pltpu.make_async_copy(
                        et_hbm.at[:, pl.ds(step * BLK + a * SLAB, BLK)],
                        dst, sems.at[buf]).start()

                lax.cond(step >= tail0, tail_cp, main_cp)
            else:
                pltpu.make_async_copy(
                    et_hbm.at[:, pl.ds(step * BLK + a * SLAB, BLK)],
                    dst, sems.at[buf]).start()

        @pl.when(i == 0)
        def _prime():
            for a in range(NSLAB):
                piece(i, 0, a)

        @pl.when(i < nstep - 1)
        def _fire_next():
            for a in range(NSLAB):
                piece(i + 1, (i + 1) % 2, a)

        for a in range(NSLAB):
            pltpu.make_async_copy(
                et_hbm.at[:, pl.ds(0, BLK)],
                scratch.at[i % 2, pl.ds(a * E, E), :],
                sems.at[i % 2],
            ).wait()

        lhs = jnp.tanh(scratch[i % 2])
        acc = lax.dot_general(lhs, w2_ref[...], (((0,), (0,)), ((), ())),
                              preferred_element_type=jnp.float32)
        out_ref[...] = acc + b2_ref[...]

    return pl.pallas_call(
        body,
        grid=(nstep,),
        in_specs=[
            pl.BlockSpec(memory_space=pltpu.MemorySpace.HBM),
            pl.BlockSpec(memory_space=pltpu.MemorySpace.HBM),
            pl.BlockSpec((K, NSLAB * PADW), lambda i: (0, 0)),
            pl.BlockSpec((1, NSLAB * PADW), lambda i: (0, 0)),
        ],
        out_specs=pl.BlockSpec((BLK, NSLAB * PADW), lambda i: (i, 0)),
        out_shape=jax.ShapeDtypeStruct((SLAB, NSLAB * PADW), jnp.float32),
        scratch_shapes=[
            pltpu.VMEM((2, K, BLK), jnp.float32),
            pltpu.SemaphoreType.DMA((2,)),
        ],
    )


@functools.lru_cache(maxsize=None)
def _make_gather(T, LBL, H, B):
    info = plsc.get_sparse_core_info()
    NW = info.num_cores * info.num_subcores
    n_chunks = T // CHUNK
    CPW = n_chunks // NW
    NG = CPW // GRP
    TPW = CPW * CHUNK
    GT = GRP * CHUNK
    assert CPW * NW * CHUNK == T and NG * GRP == CPW
    assert (B // CHUNK) % GRP == 0

    mesh = plsc.VectorSubcoreMesh(core_axis_name="c", subcore_axis_name="s")

    assert NG % 2 == 0

    @functools.partial(
        pl.kernel,
        mesh=mesh,
        out_type=jax.ShapeDtypeStruct((LBL * H * B,), jnp.float32),
        scratch_types=[
            pltpu.VMEM((TPW,), jnp.int32),
            pltpu.VMEM((TPW,), jnp.int32),
            pltpu.VMEM((GT, PADW), jnp.float32),
            pltpu.VMEM((GT, PADW), jnp.float32),

            pltpu.VMEM((LBL * GT,), jnp.float32),
            pltpu.VMEM((LBL * GT,), jnp.float32),
            pltpu.SemaphoreType.DMA,
            pltpu.SemaphoreType.DMA,
            pltpu.SemaphoreType.DMA,
            pltpu.SemaphoreType.DMA,
        ],
        compiler_params=pltpu.CompilerParams(
            use_tc_tiling_on_sc=False, needs_layout_passes=False),
    )
    def gather_kernel(p_hbm, idx_hbm, out_hbm, idx_v, midx_v,
                      rows_a, rows_b, out_a, out_b,
                      gsem_a, gsem_b, osem_a, osem_b):
        wid = lax.axis_index("s") * info.num_cores + lax.axis_index("c")
        tok0 = wid * TPW
        pltpu.sync_copy(idx_hbm.at[pl.ds(tok0, TPW)], idx_v)

        def conv(t, carry):
            v = idx_v[pl.ds(t * 16, 16)]
            m = ((v & (SLAB - 1)) << 3) | lax.shift_right_logical(v, 17)
            midx_v[pl.ds(t * 16, 16)] = m
            return carry

        lax.fori_loop(0, TPW // 16, conv, 0, unroll=8)

        iota = lax.iota(jnp.int32, 16)
        bufs = ((rows_a, gsem_a, out_a, osem_a),
                (rows_b, gsem_b, out_b, osem_b))

        def gather_fire(g, rows, gsem):
            for i in range(GRP):
                pltpu.async_copy(
                    p_hbm.at[midx_v.at[pl.ds(g * GT + i * CHUNK, CHUNK)]],
                    rows.at[pl.ds(i * CHUNK, CHUNK)],
                    gsem,
                )

        def gather_drain(g, rows, gsem):
            for i in range(GRP):
                pltpu.make_async_copy(
                    p_hbm.at[midx_v.at[pl.ds(g * GT + i * CHUNK, CHUNK)]],
                    rows.at[pl.ds(i * CHUNK, CHUNK)],
                    gsem,
                ).wait()

        def out_segments(g, out_v, osem):
            c0 = wid * CPW + g * GRP
            h = c0 // (B // CHUNK)
            b0 = (c0 % (B // CHUNK)) * CHUNK
            return [
                (out_v.at[pl.ds(j * GT, GT)],
                 out_hbm.at[pl.ds((j * H + h) * B + b0, GT)], osem)
                for j in range(LBL)
            ]

        def out_fire(g, out_v, osem):
            for src, dst, sem in out_segments(g, out_v, osem):
                pltpu.async_copy(src, dst, sem)

        def out_drain(g, out_v, osem):
            for src, dst, sem in out_segments(g, out_v, osem):
                pltpu.make_async_copy(src, dst, sem).wait()

        gather_fire(0, rows_a, gsem_a)

        def outer(g2, carry):
            for sub in range(2):
                rows, gsem, out_v, osem = bufs[sub]
                nrows, ngsem = bufs[1 - sub][0], bufs[1 - sub][1]
                g = g2 * 2 + sub

                @pl.when(g < NG - 1)
                def _fire_next():
                    gather_fire(g + 1, nrows, ngsem)

                gather_drain(g, rows, gsem)

                @pl.when(g >= 2)
                def _drain_out():
                    out_drain(g - 2, out_v, osem)

                for s in range(GT // 16):
                    ridx = iota + 16 * s
                    for j in range(LBL):
                        vals = plsc.load_gather(rows, [ridx, jnp.full((16,), j, jnp.int32)])
                        out_v[pl.ds(j * GT + 16 * s, 16)] = vals

                out_fire(g, out_v, osem)
            return carry

        lax.fori_loop(0, NG // 2, outer, 0)
        out_drain(NG - 2, out_a, osem_a)
        out_drain(NG - 1, out_b, osem_b)

    return gather_kernel


def kernel(input, emb_table, W, b):
    B, H = input.shape
    V, E = emb_table.shape
    LBL = W.shape[1]
    wpad = jnp.pad(W, ((0, 0), (0, PADW - LBL)))
    eye = jnp.eye(NSLAB, dtype=W.dtype)
    w2 = (eye[:, None, :, None] * wpad[None, :, None, :]).reshape(
        NSLAB * E, NSLAB * PADW)
    b2 = jnp.tile(jnp.pad(b, (0, PADW - LBL)), NSLAB).reshape(1, NSLAB * PADW)

    embT = emb_table.T
    tail0 = (V - (NSLAB - 1) * SLAB) // BLK
    tail_v0 = (NSLAB - 1) * SLAB + tail0 * BLK
    tail_w = SLAB - tail0 * BLK
    tail = jnp.zeros((E, tail_w), emb_table.dtype)
    tail = tail.at[:, :V - tail_v0].set(emb_table[tail_v0:, :].T)

    P3 = _make_fold(E, V)(embT, tail, w2, b2)
    P = P3.reshape(NSLAB * SLAB, PADW)

    T = B * H
    idx_flat = input.T.reshape(T)
    out_flat = _make_gather(T, LBL, H, B)(P, idx_flat)
    return out_flat.reshape(LBL, H, B).transpose(2, 1, 0)


# speedup vs baseline: 1.0038x; 1.0038x over previous
import functools

import jax
import jax.numpy as jnp
from jax import lax
from jax.experimental import pallas as pl
from jax.experimental.pallas import tpu as pltpu
from jax.experimental.pallas import tpu_sc as plsc

PADW = 16
CHUNK = 128
GRP = 4
NSLAB = 8
SLAB = 1 << 17
BLK = 1024


def _make_fold(E, V):
    K = NSLAB * E
    nstep = SLAB // BLK
    tail0 = (V - (NSLAB - 1) * SLAB) // BLK

    def body(et_hbm, tail_hbm, w2_ref, b2_ref, out_ref, scratch, sems):
        i = pl.program_id(0)

        def piece(step, buf, a):
            dst = scratch.at[buf, pl.ds(a * E, E), :]
            if a == NSLAB - 1:
                def tail_cp():
                    pltpu.make_async_copy(
                        tail_hbm.at[:, pl.ds((step - tail0) * BLK, BLK)],
                        dst, sems.at[buf]).start()

                def main_cp():
                    pltpu.make_async_copy(
                        et_hbm.at[:, pl.ds(step * BLK + a * SLAB, BLK)],
                        dst, sems.at[buf]).start()

                lax.cond(step >= tail0, tail_cp, main_cp)
            else:
                pltpu.make_async_copy(
                    et_hbm.at[:, pl.ds(step * BLK + a * SLAB, BLK)],
                    dst, sems.at[buf]).start()

        @pl.when(i == 0)
        def _prime():
            for a in range(NSLAB):
                piece(i, 0, a)

        @pl.when(i < nstep - 1)
        def _fire_next():
            for a in range(NSLAB):
                piece(i + 1, (i + 1) % 2, a)

        for a in range(NSLAB):
            pltpu.make_async_copy(
                et_hbm.at[:, pl.ds(0, BLK)],
                scratch.at[i % 2, pl.ds(a * E, E), :],
                sems.at[i % 2],
            ).wait()

        lhs = jnp.tanh(scratch[i % 2])
        acc = lax.dot_general(lhs, w2_ref[...], (((0,), (0,)), ((), ())),
                              preferred_element_type=jnp.float32)
        out_ref[...] = acc + b2_ref[...]

    return pl.pallas_call(
        body,
        grid=(nstep,),
        in_specs=[
            pl.BlockSpec(memory_space=pltpu.MemorySpace.HBM),
            pl.BlockSpec(memory_space=pltpu.MemorySpace.HBM),
            pl.BlockSpec((K, NSLAB * PADW), lambda i: (0, 0)),
            pl.BlockSpec((1, NSLAB * PADW), lambda i: (0, 0)),
        ],
        out_specs=pl.BlockSpec((BLK, NSLAB * PADW), lambda i: (i, 0)),
        out_shape=jax.ShapeDtypeStruct((SLAB, NSLAB * PADW), jnp.float32),
        scratch_shapes=[
            pltpu.VMEM((2, K, BLK), jnp.float32),
            pltpu.SemaphoreType.DMA((2,)),
        ],
    )


@functools.lru_cache(maxsize=None)
def _make_gather(T, LBL, H, B):
    info = plsc.get_sparse_core_info()
    NW = info.num_cores * info.num_subcores
    n_chunks = T // CHUNK
    CPW = n_chunks // NW
    NG = CPW // GRP
    TPW = CPW * CHUNK
    GT = GRP * CHUNK
    assert CPW * NW * CHUNK == T and NG * GRP == CPW
    assert (B // CHUNK) % GRP == 0

    mesh = plsc.VectorSubcoreMesh(core_axis_name="c", subcore_axis_name="s")

    assert NG % 2 == 0

    @functools.partial(
        pl.kernel,
        mesh=mesh,
        out_type=jax.ShapeDtypeStruct((LBL * H * B,), jnp.float32),
        scratch_types=[
            pltpu.VMEM((TPW,), jnp.int32),
            pltpu.VMEM((TPW,), jnp.int32),
            pltpu.VMEM((GT, PADW), jnp.float32),
            pltpu.VMEM((GT, PADW), jnp.float32),

            pltpu.VMEM((LBL * GT,), jnp.float32),
            pltpu.VMEM((LBL * GT,), jnp.float32),
            pltpu.SemaphoreType.DMA,
            pltpu.SemaphoreType.DMA,
            pltpu.SemaphoreType.DMA,
            pltpu.SemaphoreType.DMA,
        ],
        compiler_params=pltpu.CompilerParams(
            use_tc_tiling_on_sc=False, needs_layout_passes=False),
    )
    def gather_kernel(p_hbm, idx_hbm, out_hbm, idx_v, midx_v,
                      rows_a, rows_b, out_a, out_b,
                      gsem_a, gsem_b, osem_a, osem_b):
        wid = lax.axis_index("s") * info.num_cores + lax.axis_index("c")
        tok0 = wid * TPW
        pltpu.sync_copy(idx_hbm.at[pl.ds(tok0, TPW)], idx_v)

        def conv(t, carry):
            v = idx_v[pl.ds(t * 16, 16)]
            m = ((v & (SLAB - 1)) << 3) | lax.shift_right_logical(v, 17)
            midx_v[pl.ds(t * 16, 16)] = m
            return carry

        lax.fori_loop(0, TPW // 16, conv, 0, unroll=8)

        iota = lax.iota(jnp.int32, 16)
        bufs = ((rows_a, gsem_a, out_a, osem_a),
                (rows_b, gsem_b, out_b, osem_b))

        def gather_fire(g, rows, gsem):
            for i in range(GRP):
                pltpu.async_copy(
                    p_hbm.at[midx_v.at[pl.ds(g * GT + i * CHUNK, CHUNK)]],
                    rows.at[pl.ds(i * CHUNK, CHUNK)],
                    gsem,
                )

        def gather_drain(g, rows, gsem):
            for i in range(GRP):
                pltpu.make_async_copy(
                    p_hbm.at[midx_v.at[pl.ds(g * GT + i * CHUNK, CHUNK)]],
                    rows.at[pl.ds(i * CHUNK, CHUNK)],
                    gsem,
                ).wait()

        def out_segments(g, out_v, osem):
            c0 = wid * CPW + g * GRP
            h = c0 // (B // CHUNK)
            b0 = (c0 % (B // CHUNK)) * CHUNK
            return [
                (out_v.at[pl.ds(j * GT, GT)],
                 out_hbm.at[pl.ds((j * H + h) * B + b0, GT)], osem)
                for j in range(LBL)
            ]

        def out_fire(g, out_v, osem):
            for src, dst, sem in out_segments(g, out_v, osem):
                pltpu.async_copy(src, dst, sem)

        def out_drain(g, out_v, osem):
            for src, dst, sem in out_segments(g, out_v, osem):
                pltpu.make_async_copy(src, dst, sem).wait()

        gather_fire(0, rows_a, gsem_a)

        def outer(g2, carry):
            for sub in range(2):
                rows, gsem, out_v, osem = bufs[sub]
                nrows, ngsem = bufs[1 - sub][0], bufs[1 - sub][1]
                g = g2 * 2 + sub

                @pl.when(g < NG - 1)
                def _fire_next():
                    gather_fire(g + 1, nrows, ngsem)

                gather_drain(g, rows, gsem)

                @pl.when(g >= 2)
                def _drain_out():
                    out_drain(g - 2, out_v, osem)

                for s in range(GT // 16):
                    ridx = iota + 16 * s
                    for j in range(LBL):
                        vals = plsc.load_gather(rows, [ridx, jnp.full((16,), j, jnp.int32)])
                        out_v[pl.ds(j * GT + 16 * s, 16)] = vals

                out_fire(g, out_v, osem)
            return carry

        lax.fori_loop(0, NG // 2, outer, 0)
        out_drain(NG - 2, out_a, osem_a)
        out_drain(NG - 1, out_b, osem_b)

    return gather_kernel


def kernel(input, emb_table, W, b):
    B, H = input.shape
    V, E = emb_table.shape
    LBL = W.shape[1]
    wpad = jnp.pad(W, ((0, 0), (0, PADW - LBL)))
    eye = jnp.eye(NSLAB, dtype=W.dtype)
    w2 = (eye[:, None, :, None] * wpad[None, :, None, :]).reshape(
        NSLAB * E, NSLAB * PADW)
    b2 = jnp.tile(jnp.pad(b, (0, PADW - LBL)), NSLAB).reshape(1, NSLAB * PADW)

    embT = emb_table.T
    tail0 = (V - (NSLAB - 1) * SLAB) // BLK
    tail_v0 = (NSLAB - 1) * SLAB + tail0 * BLK
    tail_w = SLAB - tail0 * BLK
    tail = jnp.zeros((E, tail_w), emb_table.dtype)
    tail = tail.at[:, :V - tail_v0].set(emb_table[tail_v0:, :].T)

    P3 = _make_fold(E, V)(embT, tail, w2, b2)
    P = P3.reshape(NSLAB * SLAB, PADW)

    T = B * H
    idx_flat = input.T.reshape(T)
    out_flat = _make_gather(T, LBL, H, B)(P, idx_flat)
    return out_flat.reshape(LBL, H, B).transpose(2, 1, 0)


# speedup vs baseline: 1.0678x; 1.0638x over previous
import functools

import jax
import jax.numpy as jnp
from jax import lax
from jax.experimental import pallas as pl
from jax.experimental.pallas import tpu as pltpu
from jax.experimental.pallas import tpu_sc as plsc

PADW = 16
CHUNK = 128
GRP = 4
NSLAB = 8
SLAB = 1 << 17
BLK = 1024


def _make_fold(E, V):
    K = NSLAB * E
    nstep = SLAB // BLK
    tail0 = (V - (NSLAB - 1) * SLAB) // BLK

    def body(et_hbm, tail_hbm, w2_ref, b2_ref, out_ref, scratch, sems):
        i = pl.program_id(0)

        def piece(step, buf, a):
            dst = scratch.at[buf, pl.ds(a * E, E), :]
            if a == NSLAB - 1:
                def tail_cp():
                    pltpu.make_async_copy(
                        tail_hbm.at[:, pl.ds(0, BLK)], dst, sems.at[buf]).start()

                def garbage_cp():
                    pltpu.make_async_copy(
                        et_hbm.at[:, pl.ds(0, BLK)], dst, sems.at[buf]).start()

                def main_cp():
                    pltpu.make_async_copy(
                        et_hbm.at[:, pl.ds(step * BLK + a * SLAB, BLK)],
                        dst, sems.at[buf]).start()

                lax.cond(step < tail0, main_cp,
                         lambda: lax.cond(step == tail0, tail_cp, garbage_cp))
            else:
                pltpu.make_async_copy(
                    et_hbm.at[:, pl.ds(step * BLK + a * SLAB, BLK)],
                    dst, sems.at[buf]).start()

        @pl.when(i == 0)
        def _prime():
            for a in range(NSLAB):
                piece(i, 0, a)

        @pl.when(i < nstep - 1)
        def _fire_next():
            for a in range(NSLAB):
                piece(i + 1, (i + 1) % 2, a)

        for a in range(NSLAB):
            pltpu.make_async_copy(
                et_hbm.at[:, pl.ds(0, BLK)],
                scratch.at[i % 2, pl.ds(a * E, E), :],
                sems.at[i % 2],
            ).wait()

        lhs = jnp.tanh(scratch[i % 2])
        acc = lax.dot_general(lhs, w2_ref[...], (((0,), (0,)), ((), ())),
                              preferred_element_type=jnp.float32)
        out_ref[...] = acc + b2_ref[...]

    return pl.pallas_call(
        body,
        grid=(nstep,),
        in_specs=[
            pl.BlockSpec(memory_space=pltpu.MemorySpace.HBM),
            pl.BlockSpec(memory_space=pltpu.MemorySpace.HBM),
            pl.BlockSpec((K, NSLAB * PADW), lambda i: (0, 0)),
            pl.BlockSpec((1, NSLAB * PADW), lambda i: (0, 0)),
        ],
        out_specs=pl.BlockSpec((BLK, NSLAB * PADW), lambda i: (i, 0)),
        out_shape=jax.ShapeDtypeStruct((SLAB, NSLAB * PADW), jnp.float32),
        scratch_shapes=[
            pltpu.VMEM((2, K, BLK), jnp.float32),
            pltpu.SemaphoreType.DMA((2,)),
        ],
    )


@functools.lru_cache(maxsize=None)
def _make_gather(T, LBL, H, B):
    info = plsc.get_sparse_core_info()
    NW = info.num_cores * info.num_subcores
    n_chunks = T // CHUNK
    CPW = n_chunks // NW
    NG = CPW // GRP
    TPW = CPW * CHUNK
    GT = GRP * CHUNK
    assert CPW * NW * CHUNK == T and NG * GRP == CPW
    assert (B // CHUNK) % GRP == 0

    mesh = plsc.VectorSubcoreMesh(core_axis_name="c", subcore_axis_name="s")

    assert NG % 2 == 0

    @functools.partial(
        pl.kernel,
        mesh=mesh,
        out_type=jax.ShapeDtypeStruct((LBL * H * B,), jnp.float32),
        scratch_types=[
            pltpu.VMEM((TPW,), jnp.int32),
            pltpu.VMEM((TPW,), jnp.int32),
            pltpu.VMEM((GT, PADW), jnp.float32),
            pltpu.VMEM((GT, PADW), jnp.float32),

            pltpu.VMEM((LBL * GT,), jnp.float32),
            pltpu.VMEM((LBL * GT,), jnp.float32),
            pltpu.SemaphoreType.DMA,
            pltpu.SemaphoreType.DMA,
            pltpu.SemaphoreType.DMA,
            pltpu.SemaphoreType.DMA,
        ],
        compiler_params=pltpu.CompilerParams(
            use_tc_tiling_on_sc=False, needs_layout_passes=False),
    )
    def gather_kernel(p_hbm, idx_hbm, out_hbm, idx_v, midx_v,
                      rows_a, rows_b, out_a, out_b,
                      gsem_a, gsem_b, osem_a, osem_b):
        wid = lax.axis_index("s") * info.num_cores + lax.axis_index("c")
        tok0 = wid * TPW
        pltpu.sync_copy(idx_hbm.at[pl.ds(tok0, TPW)], idx_v)

        def conv_group(gg):
            for t in range(GT // 16):
                v = idx_v[pl.ds(gg * GT + t * 16, 16)]
                m = ((v & (SLAB - 1)) << 3) | lax.shift_right_logical(v, 17)
                midx_v[pl.ds(gg * GT + t * 16, 16)] = m

        iota = lax.iota(jnp.int32, 16)
        bufs = ((rows_a, gsem_a, out_a, osem_a),
                (rows_b, gsem_b, out_b, osem_b))

        def gather_fire(g, rows, gsem):
            for i in range(GRP):
                pltpu.async_copy(
                    p_hbm.at[midx_v.at[pl.ds(g * GT + i * CHUNK, CHUNK)]],
                    rows.at[pl.ds(i * CHUNK, CHUNK)],
                    gsem,
                )

        def gather_drain(g, rows, gsem):
            for i in range(GRP):
                pltpu.make_async_copy(
                    p_hbm.at[midx_v.at[pl.ds(g * GT + i * CHUNK, CHUNK)]],
                    rows.at[pl.ds(i * CHUNK, CHUNK)],
                    gsem,
                ).wait()

        def out_segments(g, out_v, osem):
            c0 = wid * CPW + g * GRP
            h = c0 // (B // CHUNK)
            b0 = (c0 % (B // CHUNK)) * CHUNK
            return [
                (out_v.at[pl.ds(j * GT, GT)],
                 out_hbm.at[pl.ds((j * H + h) * B + b0, GT)], osem)
                for j in range(LBL)
            ]

        def out_fire(g, out_v, osem):
            for src, dst, sem in out_segments(g, out_v, osem):
                pltpu.async_copy(src, dst, sem)

        def out_drain(g, out_v, osem):
            for src, dst, sem in out_segments(g, out_v, osem):
                pltpu.make_async_copy(src, dst, sem).wait()

        conv_group(0)
        conv_group(1)
        gather_fire(0, rows_a, gsem_a)

        def outer(g2, carry):
            for sub in range(2):
                rows, gsem, out_v, osem = bufs[sub]
                nrows, ngsem = bufs[1 - sub][0], bufs[1 - sub][1]
                g = g2 * 2 + sub

                @pl.when(g < NG - 1)
                def _fire_next():
                    gather_fire(g + 1, nrows, ngsem)

                @pl.when(g < NG - 2)
                def _conv_ahead():
                    conv_group(g + 2)

                gather_drain(g, rows, gsem)

                @pl.when(g >= 2)
                def _drain_out():
                    out_drain(g - 2, out_v, osem)

                for s in range(GT // 16):
                    ridx = iota + 16 * s
                    for j in range(LBL):
                        vals = plsc.load_gather(rows, [ridx, jnp.full((16,), j, jnp.int32)])
                        out_v[pl.ds(j * GT + 16 * s, 16)] = vals

                out_fire(g, out_v, osem)
            return carry

        lax.fori_loop(0, NG // 2, outer, 0)
        out_drain(NG - 2, out_a, osem_a)
        out_drain(NG - 1, out_b, osem_b)

    return gather_kernel


def kernel(input, emb_table, W, b):
    B, H = input.shape
    V, E = emb_table.shape
    LBL = W.shape[1]
    wpad = jnp.pad(W, ((0, 0), (0, PADW - LBL)))
    eye = jnp.eye(NSLAB, dtype=W.dtype)
    w2 = (eye[:, None, :, None] * wpad[None, :, None, :]).reshape(
        NSLAB * E, NSLAB * PADW)
    b2 = jnp.tile(jnp.pad(b, (0, PADW - LBL)), NSLAB).reshape(1, NSLAB * PADW)

    embT = emb_table.T
    tail0 = (V - (NSLAB - 1) * SLAB) // BLK
    tail_v0 = (NSLAB - 1) * SLAB + tail0 * BLK
    tail = jnp.zeros((E, BLK), emb_table.dtype)
    tail = tail.at[:, :V - tail_v0].set(emb_table[tail_v0:, :].T)

    P3 = _make_fold(E, V)(embT, tail, w2, b2)
    P = P3.reshape(NSLAB * SLAB, PADW)

    T = B * H
    idx_flat = input.T.reshape(T)
    out_flat = _make_gather(T, LBL, H, B)(P, idx_flat)
    return out_flat.reshape(LBL, H, B).transpose(2, 1, 0)


# speedup vs baseline: 1.2013x; 1.1249x over previous
import functools

import jax
import jax.numpy as jnp
from jax import lax
from jax.experimental import pallas as pl
from jax.experimental.pallas import tpu as pltpu
from jax.experimental.pallas import tpu_sc as plsc

PADW = 16
CHUNK = 128
GRP = 4
NSLAB = 8
SLAB = 1 << 17
BLK = 2048


def _make_fold(E, V):
    K = NSLAB * E
    nstep = SLAB // BLK
    tail0 = (V - (NSLAB - 1) * SLAB) // BLK

    def body(et_hbm, tail_hbm, w2_ref, b2_ref, out_ref, scratch, sems):
        i = pl.program_id(0)

        def piece(step, buf, a):
            dst = scratch.at[buf, pl.ds(a * E, E), :]
            if a == NSLAB - 1:
                def tail_cp():
                    pltpu.make_async_copy(
                        tail_hbm.at[:, pl.ds(0, BLK)], dst, sems.at[buf]).start()

                def garbage_cp():
                    pltpu.make_async_copy(
                        et_hbm.at[:, pl.ds(0, BLK)], dst, sems.at[buf]).start()

                def main_cp():
                    pltpu.make_async_copy(
                        et_hbm.at[:, pl.ds(step * BLK + a * SLAB, BLK)],
                        dst, sems.at[buf]).start()

                lax.cond(step < tail0, main_cp,
                         lambda: lax.cond(step == tail0, tail_cp, garbage_cp))
            else:
                pltpu.make_async_copy(
                    et_hbm.at[:, pl.ds(step * BLK + a * SLAB, BLK)],
                    dst, sems.at[buf]).start()

        @pl.when(i == 0)
        def _prime():
            for a in range(NSLAB):
                piece(i, 0, a)

        @pl.when(i < nstep - 1)
        def _fire_next():
            for a in range(NSLAB):
                piece(i + 1, (i + 1) % 2, a)

        for a in range(NSLAB):
            pltpu.make_async_copy(
                et_hbm.at[:, pl.ds(0, BLK)],
                scratch.at[i % 2, pl.ds(a * E, E), :],
                sems.at[i % 2],
            ).wait()

        lhs = jnp.tanh(scratch[i % 2])
        acc = lax.dot_general(lhs, w2_ref[...], (((0,), (0,)), ((), ())),
                              preferred_element_type=jnp.float32)
        out_ref[...] = acc + b2_ref[...]

    return pl.pallas_call(
        body,
        grid=(nstep,),
        in_specs=[
            pl.BlockSpec(memory_space=pltpu.MemorySpace.HBM),
            pl.BlockSpec(memory_space=pltpu.MemorySpace.HBM),
            pl.BlockSpec((K, NSLAB * PADW), lambda i: (0, 0)),
            pl.BlockSpec((1, NSLAB * PADW), lambda i: (0, 0)),
        ],
        out_specs=pl.BlockSpec((BLK, NSLAB * PADW), lambda i: (i, 0)),
        out_shape=jax.ShapeDtypeStruct((SLAB, NSLAB * PADW), jnp.float32),
        scratch_shapes=[
            pltpu.VMEM((2, K, BLK), jnp.float32),
            pltpu.SemaphoreType.DMA((2,)),
        ],
    )


@functools.lru_cache(maxsize=None)
def _make_gather(T, LBL, H, B):
    info = plsc.get_sparse_core_info()
    NW = info.num_cores * info.num_subcores
    n_chunks = T // CHUNK
    CPW = n_chunks // NW
    NG = CPW // GRP
    TPW = CPW * CHUNK
    GT = GRP * CHUNK
    assert CPW * NW * CHUNK == T and NG * GRP == CPW
    assert (B // CHUNK) % GRP == 0

    mesh = plsc.VectorSubcoreMesh(core_axis_name="c", subcore_axis_name="s")

    assert NG % 2 == 0

    @functools.partial(
        pl.kernel,
        mesh=mesh,
        out_type=jax.ShapeDtypeStruct((LBL * H * B,), jnp.float32),
        scratch_types=[
            pltpu.VMEM((TPW,), jnp.int32),
            pltpu.VMEM((TPW,), jnp.int32),
            pltpu.VMEM((GT, PADW), jnp.float32),
            pltpu.VMEM((GT, PADW), jnp.float32),

            pltpu.VMEM((LBL * GT,), jnp.float32),
            pltpu.VMEM((LBL * GT,), jnp.float32),
            pltpu.SemaphoreType.DMA,
            pltpu.SemaphoreType.DMA,
            pltpu.SemaphoreType.DMA,
            pltpu.SemaphoreType.DMA,
        ],
        compiler_params=pltpu.CompilerParams(
            use_tc_tiling_on_sc=False, needs_layout_passes=False),
    )
    def gather_kernel(p_hbm, idx_hbm, out_hbm, idx_v, midx_v,
                      rows_a, rows_b, out_a, out_b,
                      gsem_a, gsem_b, osem_a, osem_b):
        wid = lax.axis_index("s") * info.num_cores + lax.axis_index("c")
        tok0 = wid * TPW
        pltpu.sync_copy(idx_hbm.at[pl.ds(tok0, TPW)], idx_v)

        def conv_group(gg):
            for t in range(GT // 16):
                v = idx_v[pl.ds(gg * GT + t * 16, 16)]
                m = ((v & (SLAB - 1)) << 3) | lax.shift_right_logical(v, 17)
                midx_v[pl.ds(gg * GT + t * 16, 16)] = m

        iota = lax.iota(jnp.int32, 16)
        bufs = ((rows_a, gsem_a, out_a, osem_a),
                (rows_b, gsem_b, out_b, osem_b))

        def gather_fire(g, rows, gsem):
            for i in range(GRP):
                pltpu.async_copy(
                    p_hbm.at[midx_v.at[pl.ds(g * GT + i * CHUNK, CHUNK)]],
                    rows.at[pl.ds(i * CHUNK, CHUNK)],
                    gsem,
                )

        def gather_drain(g, rows, gsem):
            for i in range(GRP):
                pltpu.make_async_copy(
                    p_hbm.at[midx_v.at[pl.ds(g * GT + i * CHUNK, CHUNK)]],
                    rows.at[pl.ds(i * CHUNK, CHUNK)],
                    gsem,
                ).wait()

        def out_segments(g, out_v, osem):
            c0 = wid * CPW + g * GRP
            h = c0 // (B // CHUNK)
            b0 = (c0 % (B // CHUNK)) * CHUNK
            return [
                (out_v.at[pl.ds(j * GT, GT)],
                 out_hbm.at[pl.ds((j * H + h) * B + b0, GT)], osem)
                for j in range(LBL)
            ]

        def out_fire(g, out_v, osem):
            for src, dst, sem in out_segments(g, out_v, osem):
                pltpu.async_copy(src, dst, sem)

        def out_drain(g, out_v, osem):
            for src, dst, sem in out_segments(g, out_v, osem):
                pltpu.make_async_copy(src, dst, sem).wait()

        conv_group(0)
        conv_group(1)
        gather_fire(0, rows_a, gsem_a)

        def outer(g2, carry):
            for sub in range(2):
                rows, gsem, out_v, osem = bufs[sub]
                nrows, ngsem = bufs[1 - sub][0], bufs[1 - sub][1]
                g = g2 * 2 + sub

                @pl.when(g < NG - 1)
                def _fire_next():
                    gather_fire(g + 1, nrows, ngsem)

                @pl.when(g < NG - 2)
                def _conv_ahead():
                    conv_group(g + 2)

                gather_drain(g, rows, gsem)

                @pl.when(g >= 2)
                def _drain_out():
                    out_drain(g - 2, out_v, osem)

                for s in range(GT // 16):
                    ridx = iota + 16 * s
                    for j in range(LBL):
                        vals = plsc.load_gather(rows, [ridx, jnp.full((16,), j, jnp.int32)])
                        out_v[pl.ds(j * GT + 16 * s, 16)] = vals

                out_fire(g, out_v, osem)
            return carry

        lax.fori_loop(0, NG // 2, outer, 0)
        out_drain(NG - 2, out_a, osem_a)
        out_drain(NG - 1, out_b, osem_b)

    return gather_kernel


def kernel(input, emb_table, W, b):
    B, H = input.shape
    V, E = emb_table.shape
    LBL = W.shape[1]
    wpad = jnp.pad(W, ((0, 0), (0, PADW - LBL)))
    eye = jnp.eye(NSLAB, dtype=W.dtype)
    w2 = (eye[:, None, :, None] * wpad[None, :, None, :]).reshape(
        NSLAB * E, NSLAB * PADW)
    b2 = jnp.tile(jnp.pad(b, (0, PADW - LBL)), NSLAB).reshape(1, NSLAB * PADW)

    embT = emb_table.T
    tail0 = (V - (NSLAB - 1) * SLAB) // BLK
    tail_v0 = (NSLAB - 1) * SLAB + tail0 * BLK
    tail = jnp.zeros((E, BLK), emb_table.dtype)
    tail = tail.at[:, :V - tail_v0].set(emb_table[tail_v0:, :].T)

    P3 = _make_fold(E, V)(embT, tail, w2, b2)
    P = P3.reshape(NSLAB * SLAB, PADW)

    T = B * H
    idx_flat = input.T.reshape(T)
    out_flat = _make_gather(T, LBL, H, B)(P, idx_flat)
    return out_flat.reshape(LBL, H, B).transpose(2, 1, 0)


# speedup vs baseline: 1.2591x; 1.0481x over previous
import functools

import jax
import jax.numpy as jnp
from jax import lax
from jax.experimental import pallas as pl
from jax.experimental.pallas import tpu as pltpu
from jax.experimental.pallas import tpu_sc as plsc

PADW = 16
CHUNK = 128
GRP = 4
NSLAB = 8
SLAB = 1 << 17
BLK = 4096


def _make_fold(E, V):
    K = NSLAB * E
    nstep = SLAB // BLK
    tail0 = (V - (NSLAB - 1) * SLAB) // BLK

    def body(et_hbm, tail_hbm, w2_ref, b2_ref, out_ref, scratch, sems):
        i = pl.program_id(0)

        def piece(step, buf, a):
            dst = scratch.at[buf, pl.ds(a * E, E), :]
            if a == NSLAB - 1:
                def tail_cp():
                    pltpu.make_async_copy(
                        tail_hbm.at[:, pl.ds(0, BLK)], dst, sems.at[buf]).start()

                def garbage_cp():
                    pltpu.make_async_copy(
                        et_hbm.at[:, pl.ds(0, BLK)], dst, sems.at[buf]).start()

                def main_cp():
                    pltpu.make_async_copy(
                        et_hbm.at[:, pl.ds(step * BLK + a * SLAB, BLK)],
                        dst, sems.at[buf]).start()

                lax.cond(step < tail0, main_cp,
                         lambda: lax.cond(step == tail0, tail_cp, garbage_cp))
            else:
                pltpu.make_async_copy(
                    et_hbm.at[:, pl.ds(step * BLK + a * SLAB, BLK)],
                    dst, sems.at[buf]).start()

        @pl.when(i == 0)
        def _prime():
            for a in range(NSLAB):
                piece(i, 0, a)

        @pl.when(i < nstep - 1)
        def _fire_next():
            for a in range(NSLAB):
                piece(i + 1, (i + 1) % 2, a)

        for a in range(NSLAB):
            pltpu.make_async_copy(
                et_hbm.at[:, pl.ds(0, BLK)],
                scratch.at[i % 2, pl.ds(a * E, E), :],
                sems.at[i % 2],
            ).wait()

        lhs = jnp.tanh(scratch[i % 2])
        acc = lax.dot_general(lhs, w2_ref[...], (((0,), (0,)), ((), ())),
                              preferred_element_type=jnp.float32)
        out_ref[...] = acc + b2_ref[...]

    return pl.pallas_call(
        body,
        grid=(nstep,),
        in_specs=[
            pl.BlockSpec(memory_space=pltpu.MemorySpace.HBM),
            pl.BlockSpec(memory_space=pltpu.MemorySpace.HBM),
            pl.BlockSpec((K, NSLAB * PADW), lambda i: (0, 0)),
            pl.BlockSpec((1, NSLAB * PADW), lambda i: (0, 0)),
        ],
        out_specs=pl.BlockSpec((BLK, NSLAB * PADW), lambda i: (i, 0)),
        out_shape=jax.ShapeDtypeStruct((SLAB, NSLAB * PADW), jnp.float32),
        scratch_shapes=[
            pltpu.VMEM((2, K, BLK), jnp.float32),
            pltpu.SemaphoreType.DMA((2,)),
        ],
    )


@functools.lru_cache(maxsize=None)
def _make_gather(T, LBL, H, B):
    info = plsc.get_sparse_core_info()
    NW = info.num_cores * info.num_subcores
    n_chunks = T // CHUNK
    CPW = n_chunks // NW
    NG = CPW // GRP
    TPW = CPW * CHUNK
    GT = GRP * CHUNK
    assert CPW * NW * CHUNK == T and NG * GRP == CPW
    assert (B // CHUNK) % GRP == 0

    mesh = plsc.VectorSubcoreMesh(core_axis_name="c", subcore_axis_name="s")

    assert NG % 2 == 0

    @functools.partial(
        pl.kernel,
        mesh=mesh,
        out_type=jax.ShapeDtypeStruct((LBL * H * B,), jnp.float32),
        scratch_types=[
            pltpu.VMEM((TPW,), jnp.int32),
            pltpu.VMEM((TPW,), jnp.int32),
            pltpu.VMEM((GT, PADW), jnp.float32),
            pltpu.VMEM((GT, PADW), jnp.float32),

            pltpu.VMEM((LBL * GT,), jnp.float32),
            pltpu.VMEM((LBL * GT,), jnp.float32),
            pltpu.SemaphoreType.DMA,
            pltpu.SemaphoreType.DMA,
            pltpu.SemaphoreType.DMA,
            pltpu.SemaphoreType.DMA,
        ],
        compiler_params=pltpu.CompilerParams(
            use_tc_tiling_on_sc=False, needs_layout_passes=False),
    )
    def gather_kernel(p_hbm, idx_hbm, out_hbm, idx_v, midx_v,
                      rows_a, rows_b, out_a, out_b,
                      gsem_a, gsem_b, osem_a, osem_b):
        wid = lax.axis_index("s") * info.num_cores + lax.axis_index("c")
        tok0 = wid * TPW
        pltpu.sync_copy(idx_hbm.at[pl.ds(tok0, TPW)], idx_v)

        def conv_group(gg):
            for t in range(GT // 16):
                v = idx_v[pl.ds(gg * GT + t * 16, 16)]
                m = ((v & (SLAB - 1)) << 3) | lax.shift_right_logical(v, 17)
                midx_v[pl.ds(gg * GT + t * 16, 16)] = m

        iota = lax.iota(jnp.int32, 16)
        bufs = ((rows_a, gsem_a, out_a, osem_a),
                (rows_b, gsem_b, out_b, osem_b))

        def gather_fire(g, rows, gsem):
            for i in range(GRP):
                pltpu.async_copy(
                    p_hbm.at[midx_v.at[pl.ds(g * GT + i * CHUNK, CHUNK)]],
                    rows.at[pl.ds(i * CHUNK, CHUNK)],
                    gsem,
                )

        def gather_drain(g, rows, gsem):
            for i in range(GRP):
                pltpu.make_async_copy(
                    p_hbm.at[midx_v.at[pl.ds(g * GT + i * CHUNK, CHUNK)]],
                    rows.at[pl.ds(i * CHUNK, CHUNK)],
                    gsem,
                ).wait()

        def out_segments(g, out_v, osem):
            c0 = wid * CPW + g * GRP
            h = c0 // (B // CHUNK)
            b0 = (c0 % (B // CHUNK)) * CHUNK
            return [
                (out_v.at[pl.ds(j * GT, GT)],
                 out_hbm.at[pl.ds((j * H + h) * B + b0, GT)], osem)
                for j in range(LBL)
            ]

        def out_fire(g, out_v, osem):
            for src, dst, sem in out_segments(g, out_v, osem):
                pltpu.async_copy(src, dst, sem)

        def out_drain(g, out_v, osem):
            for src, dst, sem in out_segments(g, out_v, osem):
                pltpu.make_async_copy(src, dst, sem).wait()

        conv_group(0)
        conv_group(1)
        gather_fire(0, rows_a, gsem_a)

        def outer(g2, carry):
            for sub in range(2):
                rows, gsem, out_v, osem = bufs[sub]
                nrows, ngsem = bufs[1 - sub][0], bufs[1 - sub][1]
                g = g2 * 2 + sub

                @pl.when(g < NG - 1)
                def _fire_next():
                    gather_fire(g + 1, nrows, ngsem)

                @pl.when(g < NG - 2)
                def _conv_ahead():
                    conv_group(g + 2)

                gather_drain(g, rows, gsem)

                @pl.when(g >= 2)
                def _drain_out():
                    out_drain(g - 2, out_v, osem)

                for s in range(GT // 16):
                    ridx = iota + 16 * s
                    for j in range(LBL):
                        vals = plsc.load_gather(rows, [ridx, jnp.full((16,), j, jnp.int32)])
                        out_v[pl.ds(j * GT + 16 * s, 16)] = vals

                out_fire(g, out_v, osem)
            return carry

        lax.fori_loop(0, NG // 2, outer, 0)
        out_drain(NG - 2, out_a, osem_a)
        out_drain(NG - 1, out_b, osem_b)

    return gather_kernel


def kernel(input, emb_table, W, b):
    B, H = input.shape
    V, E = emb_table.shape
    LBL = W.shape[1]
    wpad = jnp.pad(W, ((0, 0), (0, PADW - LBL)))
    eye = jnp.eye(NSLAB, dtype=W.dtype)
    w2 = (eye[:, None, :, None] * wpad[None, :, None, :]).reshape(
        NSLAB * E, NSLAB * PADW)
    b2 = jnp.tile(jnp.pad(b, (0, PADW - LBL)), NSLAB).reshape(1, NSLAB * PADW)

    embT = emb_table.T
    tail0 = (V - (NSLAB - 1) * SLAB) // BLK
    tail_v0 = (NSLAB - 1) * SLAB + tail0 * BLK
    tail = jnp.zeros((E, BLK), emb_table.dtype)
    tail = tail.at[:, :V - tail_v0].set(emb_table[tail_v0:, :].T)

    P3 = _make_fold(E, V)(embT, tail, w2, b2)
    P = P3.reshape(NSLAB * SLAB, PADW)

    T = B * H
    idx_flat = input.T.reshape(T)
    out_flat = _make_gather(T, LBL, H, B)(P, idx_flat)
    return out_flat.reshape(LBL, H, B).transpose(2, 1, 0)


# speedup vs baseline: 1.2607x; 1.0013x over previous
import functools

import jax
import jax.numpy as jnp
from jax import lax
from jax.experimental import pallas as pl
from jax.experimental.pallas import tpu as pltpu
from jax.experimental.pallas import tpu_sc as plsc

PADW = 16
CHUNK = 128
GRP = 4
NSLAB = 8
SLAB = 1 << 17
BLK = 8192


def _make_fold(E, V):
    K = NSLAB * E
    nstep = SLAB // BLK
    tail0 = (V - (NSLAB - 1) * SLAB) // BLK

    def body(et_hbm, tail_hbm, w2_ref, b2_ref, out_ref, scratch, sems):
        i = pl.program_id(0)

        def piece(step, buf, a):
            dst = scratch.at[buf, pl.ds(a * E, E), :]
            if a == NSLAB - 1:
                def tail_cp():
                    pltpu.make_async_copy(
                        tail_hbm.at[:, pl.ds(0, BLK)], dst, sems.at[buf]).start()

                def garbage_cp():
                    pltpu.make_async_copy(
                        et_hbm.at[:, pl.ds(0, BLK)], dst, sems.at[buf]).start()

                def main_cp():
                    pltpu.make_async_copy(
                        et_hbm.at[:, pl.ds(step * BLK + a * SLAB, BLK)],
                        dst, sems.at[buf]).start()

                lax.cond(step < tail0, main_cp,
                         lambda: lax.cond(step == tail0, tail_cp, garbage_cp))
            else:
                pltpu.make_async_copy(
                    et_hbm.at[:, pl.ds(step * BLK + a * SLAB, BLK)],
                    dst, sems.at[buf]).start()

        @pl.when(i == 0)
        def _prime():
            for a in range(NSLAB):
                piece(i, 0, a)

        @pl.when(i < nstep - 1)
        def _fire_next():
            for a in range(NSLAB):
                piece(i + 1, (i + 1) % 2, a)

        for a in range(NSLAB):
            pltpu.make_async_copy(
                et_hbm.at[:, pl.ds(0, BLK)],
                scratch.at[i % 2, pl.ds(a * E, E), :],
                sems.at[i % 2],
            ).wait()

        lhs = jnp.tanh(scratch[i % 2])
        acc = lax.dot_general(lhs, w2_ref[...], (((0,), (0,)), ((), ())),
                              preferred_element_type=jnp.float32)
        out_ref[...] = acc + b2_ref[...]

    return pl.pallas_call(
        body,
        grid=(nstep,),
        in_specs=[
            pl.BlockSpec(memory_space=pltpu.MemorySpace.HBM),
            pl.BlockSpec(memory_space=pltpu.MemorySpace.HBM),
            pl.BlockSpec((K, NSLAB * PADW), lambda i: (0, 0)),
            pl.BlockSpec((1, NSLAB * PADW), lambda i: (0, 0)),
        ],
        out_specs=pl.BlockSpec((BLK, NSLAB * PADW), lambda i: (i, 0)),
        out_shape=jax.ShapeDtypeStruct((SLAB, NSLAB * PADW), jnp.float32),
        scratch_shapes=[
            pltpu.VMEM((2, K, BLK), jnp.float32),
            pltpu.SemaphoreType.DMA((2,)),
        ],
    )


@functools.lru_cache(maxsize=None)
def _make_gather(T, LBL, H, B):
    info = plsc.get_sparse_core_info()
    NW = info.num_cores * info.num_subcores
    n_chunks = T // CHUNK
    CPW = n_chunks // NW
    NG = CPW // GRP
    TPW = CPW * CHUNK
    GT = GRP * CHUNK
    assert CPW * NW * CHUNK == T and NG * GRP == CPW
    assert (B // CHUNK) % GRP == 0

    mesh = plsc.VectorSubcoreMesh(core_axis_name="c", subcore_axis_name="s")

    assert NG % 2 == 0

    @functools.partial(
        pl.kernel,
        mesh=mesh,
        out_type=jax.ShapeDtypeStruct((LBL * H * B,), jnp.float32),
        scratch_types=[
            pltpu.VMEM((TPW,), jnp.int32),
            pltpu.VMEM((TPW,), jnp.int32),
            pltpu.VMEM((GT, PADW), jnp.float32),
            pltpu.VMEM((GT, PADW), jnp.float32),

            pltpu.VMEM((LBL * GT,), jnp.float32),
            pltpu.VMEM((LBL * GT,), jnp.float32),
            pltpu.SemaphoreType.DMA,
            pltpu.SemaphoreType.DMA,
            pltpu.SemaphoreType.DMA,
            pltpu.SemaphoreType.DMA,
        ],
        compiler_params=pltpu.CompilerParams(
            use_tc_tiling_on_sc=False, needs_layout_passes=False),
    )
    def gather_kernel(p_hbm, idx_hbm, out_hbm, idx_v, midx_v,
                      rows_a, rows_b, out_a, out_b,
                      gsem_a, gsem_b, osem_a, osem_b):
        wid = lax.axis_index("s") * info.num_cores + lax.axis_index("c")
        tok0 = wid * TPW
        pltpu.sync_copy(idx_hbm.at[pl.ds(tok0, TPW)], idx_v)

        def conv_group(gg):
            for t in range(GT // 16):
                v = idx_v[pl.ds(gg * GT + t * 16, 16)]
                m = ((v & (SLAB - 1)) << 3) | lax.shift_right_logical(v, 17)
                midx_v[pl.ds(gg * GT + t * 16, 16)] = m

        iota = lax.iota(jnp.int32, 16)
        bufs = ((rows_a, gsem_a, out_a, osem_a),
                (rows_b, gsem_b, out_b, osem_b))

        def gather_fire(g, rows, gsem):
            for i in range(GRP):
                pltpu.async_copy(
                    p_hbm.at[midx_v.at[pl.ds(g * GT + i * CHUNK, CHUNK)]],
                    rows.at[pl.ds(i * CHUNK, CHUNK)],
                    gsem,
                )

        def gather_drain(g, rows, gsem):
            for i in range(GRP):
                pltpu.make_async_copy(
                    p_hbm.at[midx_v.at[pl.ds(g * GT + i * CHUNK, CHUNK)]],
                    rows.at[pl.ds(i * CHUNK, CHUNK)],
                    gsem,
                ).wait()

        def out_segments(g, out_v, osem):
            c0 = wid * CPW + g * GRP
            h = c0 // (B // CHUNK)
            b0 = (c0 % (B // CHUNK)) * CHUNK
            return [
                (out_v.at[pl.ds(j * GT, GT)],
                 out_hbm.at[pl.ds((j * H + h) * B + b0, GT)], osem)
                for j in range(LBL)
            ]

        def out_fire(g, out_v, osem):
            for src, dst, sem in out_segments(g, out_v, osem):
                pltpu.async_copy(src, dst, sem)

        def out_drain(g, out_v, osem):
            for src, dst, sem in out_segments(g, out_v, osem):
                pltpu.make_async_copy(src, dst, sem).wait()

        conv_group(0)
        conv_group(1)
        gather_fire(0, rows_a, gsem_a)

        def outer(g2, carry):
            for sub in range(2):
                rows, gsem, out_v, osem = bufs[sub]
                nrows, ngsem = bufs[1 - sub][0], bufs[1 - sub][1]
                g = g2 * 2 + sub

                @pl.when(g < NG - 1)
                def _fire_next():
                    gather_fire(g + 1, nrows, ngsem)

                @pl.when(g < NG - 2)
                def _conv_ahead():
                    conv_group(g + 2)

                gather_drain(g, rows, gsem)

                @pl.when(g >= 2)
                def _drain_out():
                    out_drain(g - 2, out_v, osem)

                for s in range(GT // 16):
                    ridx = iota + 16 * s
                    for j in range(LBL):
                        vals = plsc.load_gather(rows, [ridx, jnp.full((16,), j, jnp.int32)])
                        out_v[pl.ds(j * GT + 16 * s, 16)] = vals

                out_fire(g, out_v, osem)
            return carry

        lax.fori_loop(0, NG // 2, outer, 0)
        out_drain(NG - 2, out_a, osem_a)
        out_drain(NG - 1, out_b, osem_b)

    return gather_kernel


def kernel(input, emb_table, W, b):
    B, H = input.shape
    V, E = emb_table.shape
    LBL = W.shape[1]
    wpad = jnp.pad(W, ((0, 0), (0, PADW - LBL)))
    eye = jnp.eye(NSLAB, dtype=W.dtype)
    w2 = (eye[:, None, :, None] * wpad[None, :, None, :]).reshape(
        NSLAB * E, NSLAB * PADW)
    b2 = jnp.tile(jnp.pad(b, (0, PADW - LBL)), NSLAB).reshape(1, NSLAB * PADW)

    embT = emb_table.T
    tail0 = (V - (NSLAB - 1) * SLAB) // BLK
    tail_v0 = (NSLAB - 1) * SLAB + tail0 * BLK
    tail = jnp.zeros((E, BLK), emb_table.dtype)
    tail = tail.at[:, :V - tail_v0].set(emb_table[tail_v0:, :].T)

    P3 = _make_fold(E, V)(embT, tail, w2, b2)
    P = P3.reshape(NSLAB * SLAB, PADW)

    T = B * H
    idx_flat = input.T.reshape(T)
    out_flat = _make_gather(T, LBL, H, B)(P, idx_flat)
    return out_flat.reshape(LBL, H, B).transpose(2, 1, 0)
